# baseline (speedup 1.0000x reference)
import functools

import jax
import jax.numpy as jnp
import numpy as np
from jax import lax
from jax.experimental import pallas as pl
from jax.experimental.pallas import tpu as pltpu
from jax.experimental.pallas import tpu_sc as plsc

N = 10000
E = 320000
D = 128

NC = 2
NS = 16
NW = NC * NS

NP = 10240
RPT = NP // NS
CH = 64
EPW = E // NW
PADW = 240
IBK = 20
NBLK = (EPW + PADW) // (IBK * CH)
NB = 4

DCH = 80
NCH = EPW // DCH

_MESH = plsc.VectorSubcoreMesh(
    core_axis_name="c", subcore_axis_name="s", num_cores=NC, num_subcores=NS
)


@functools.partial(
    pl.kernel,
    out_type=jax.ShapeDtypeStruct((NC, NP), jnp.float32),
    mesh=_MESH,
    scratch_types=[
        pltpu.VMEM_SHARED((NP,), jnp.float32),
        pltpu.VMEM((NCH, DCH), jnp.int32),
        pltpu.VMEM((DCH,), jnp.float32),
        pltpu.VMEM((RPT,), jnp.float32),
    ],
)
def _deg_pass(dst_hbm, out_hbm, deg_sh, idx_v, ones_v, zero_v):
    c = lax.axis_index("c")
    s = lax.axis_index("s")
    w = c * NS + s

    def fill(i, _):
        zero_v[pl.ds(i * 16, 16)] = jnp.zeros((16,), jnp.float32)
        return ()

    lax.fori_loop(0, RPT // 16, fill, ())
    for j in range(DCH // 16):
        ones_v[pl.ds(j * 16, 16)] = jnp.ones((16,), jnp.float32)

    pltpu.sync_copy(dst_hbm.at[w], idx_v)
    pltpu.sync_copy(zero_v, deg_sh.at[pl.ds(s * RPT, RPT)])
    plsc.subcore_barrier()

    def body(g, _):
        pltpu.sync_copy(ones_v, deg_sh.at[idx_v.at[g]], add=True)
        return ()

    lax.fori_loop(0, NCH, body, ())
    plsc.subcore_barrier()
    pltpu.sync_copy(deg_sh.at[pl.ds(s * RPT, RPT)],
                    out_hbm.at[c, pl.ds(s * RPT, RPT)])


@functools.partial(
    pl.kernel,
    out_type=jax.ShapeDtypeStruct((NC, NP, D), jnp.float32),
    mesh=_MESH,
    scratch_types=[
        pltpu.VMEM_SHARED((NP, D), jnp.float32),
        pltpu.VMEM((IBK, CH), jnp.int32),
        pltpu.VMEM((IBK, CH), jnp.int32),
        [pltpu.VMEM((CH, D), jnp.float32)] * NB,
        pltpu.VMEM((32, D), jnp.float32),
        [pltpu.SemaphoreType.DMA] * NB,
        [pltpu.SemaphoreType.DMA] * NB,
    ],
)
def _edge_pass(hs_hbm, src_hbm, dst_hbm, out_hbm,
               acc_sh, srcb, dstb, rows, zv, gsem, ssem):
    c = lax.axis_index("c")
    s = lax.axis_index("s")
    w = c * NS + s

    def fill(i, _):
        for j in range(D // 16):
            zv[i, pl.ds(j * 16, 16)] = jnp.zeros((16,), jnp.float32)
        return ()

    lax.fori_loop(0, 32, fill, ())

    def zacc(k, _):
        pltpu.sync_copy(zv, acc_sh.at[pl.ds(s * RPT + k * 32, 32)])
        return ()

    lax.fori_loop(0, RPT // 32, zacc, ())
    plsc.subcore_barrier()

    def blk(bk, _):
        pltpu.sync_copy(src_hbm.at[w, bk], srcb)
        pltpu.sync_copy(dst_hbm.at[w, bk], dstb)

        for b in range(NB - 1):
            pltpu.async_copy(hs_hbm.at[srcb.at[b]], rows[b], gsem[b])

        def grp(t, _):
            scat = [None] * NB
            for b in range(NB):
                j = t * NB + b
                pltpu.make_async_copy(
                    hs_hbm.at[srcb.at[j]], rows[b], gsem[b]).wait()
                scat[b] = pltpu.async_copy(
                    rows[b], acc_sh.at[dstb.at[j]], ssem[b], add=True)
                nb = (b + NB - 1) % NB
                if b == 0:
                    @pl.when(t > 0)
                    def _():
                        pltpu.make_async_copy(
                            rows[nb], acc_sh.at[dstb.at[0]], ssem[nb]).wait()

                    pltpu.async_copy(
                        hs_hbm.at[srcb.at[j + NB - 1]], rows[nb], gsem[nb])
                else:
                    prev = scat[b - 1]

                    @pl.when(t < IBK // NB - 1)
                    def _():
                        prev.wait()
                        pltpu.async_copy(
                            hs_hbm.at[srcb.at[j + NB - 1]], rows[nb],
                            gsem[nb])
            return ()

        lax.fori_loop(0, IBK // NB, grp, ())
        for b in range(NB):
            pltpu.make_async_copy(
                rows[b], acc_sh.at[dstb.at[0]], ssem[b]).wait()
        return ()

    lax.fori_loop(0, NBLK, blk, ())
    plsc.subcore_barrier()
    pltpu.sync_copy(acc_sh.at[pl.ds(s * RPT, RPT)],
                    out_hbm.at[c, pl.ds(s * RPT, RPT)])


RB = 512


def _dinv(d0, d1):
    return lax.rsqrt(d0 + d1 + 1.0)


def _prep_body(x_ref, w_ref, d0_ref, d1_ref, o_ref):
    dinv = _dinv(d0_ref[...], d1_ref[...])
    o_ref[...] = jnp.dot(x_ref[...], w_ref[...],
                         preferred_element_type=jnp.float32) * dinv


def _mid_body(a0_ref, a1_ref, hs_ref, d0_ref, d1_ref, b_ref, w_ref, o_ref):
    dinv = _dinv(d0_ref[...], d1_ref[...])
    t = (a0_ref[...] + a1_ref[...] + hs_ref[...]) * dinv + b_ref[...]
    x2 = jnp.maximum(t, 0.0)
    o_ref[...] = jnp.dot(x2, w_ref[...],
                         preferred_element_type=jnp.float32) * dinv


def _final_body(a0_ref, a1_ref, hs_ref, d0_ref, d1_ref, b_ref, o_ref):
    dinv = _dinv(d0_ref[...], d1_ref[...])
    t = (a0_ref[...] + a1_ref[...] + hs_ref[...]) * dinv + b_ref[...]
    o_ref[...] = jnp.maximum(t, 0.0)


_row_spec = pl.BlockSpec((RB, D), lambda i: (i, 0))
_col_spec = pl.BlockSpec((RB, 1), lambda i: (i, 0))
_mat_spec = pl.BlockSpec((D, D), lambda i: (0, 0))
_bias_spec = pl.BlockSpec((1, D), lambda i: (0, 0))
_out_row = jax.ShapeDtypeStruct((NP, D), jnp.float32)

_prep = pl.pallas_call(
    _prep_body,
    grid=(NP // RB,),
    in_specs=[_row_spec, _mat_spec, _col_spec, _col_spec],
    out_specs=_row_spec,
    out_shape=_out_row,
)

_mid = pl.pallas_call(
    _mid_body,
    grid=(NP // RB,),
    in_specs=[_row_spec, _row_spec, _row_spec, _col_spec, _col_spec,
              _bias_spec, _mat_spec],
    out_specs=_row_spec,
    out_shape=_out_row,
)

_final = pl.pallas_call(
    _final_body,
    grid=(NP // RB,),
    in_specs=[_row_spec, _row_spec, _row_spec, _col_spec, _col_spec,
              _bias_spec],
    out_specs=_row_spec,
    out_shape=_out_row,
)

_PAD_SRC = np.asarray(
    (np.arange(NW * PADW, dtype=np.int64) * 41) % N, np.int32
).reshape(NW, PADW)
_PAD_DST = np.asarray(
    N + (np.arange(NW * PADW, dtype=np.int64) * 7) % (NP - N), np.int32
).reshape(NW, PADW)


def kernel(x, edge_index, W1, b1, W2, b2):
    src_r = jnp.concatenate(
        [edge_index[0].reshape(NW, EPW), jnp.asarray(_PAD_SRC)], axis=1
    ).reshape(NW, NBLK, IBK, CH)
    dst_r = jnp.concatenate(
        [edge_index[1].reshape(NW, EPW), jnp.asarray(_PAD_DST)], axis=1
    ).reshape(NW, NBLK, IBK, CH)
    dst_deg = edge_index[1].reshape(NW, NCH, DCH)
    x_pad = jnp.concatenate(
        [x, jnp.zeros((NP - N, D), jnp.float32)], axis=0)

    degp = _deg_pass(dst_deg)
    d0 = degp[0].reshape(NP, 1)
    d1 = degp[1].reshape(NP, 1)

    hs1 = _prep(x_pad, W1, d0, d1)
    acc1 = _edge_pass(hs1, src_r, dst_r)
    hs2 = _mid(acc1[0], acc1[1], hs1, d0, d1, b1.reshape(1, D), W2)
    acc2 = _edge_pass(hs2, src_r, dst_r)
    out = _final(acc2[0], acc2[1], hs2, d0, d1, b2.reshape(1, D))
    return out[:N]


# speedup vs baseline: 1.0482x; 1.0482x over previous
import functools

import jax
import jax.numpy as jnp
import numpy as np
from jax import lax
from jax.experimental import pallas as pl
from jax.experimental.pallas import tpu as pltpu
from jax.experimental.pallas import tpu_sc as plsc

N = 10000
E = 320000
D = 128

NC = 2
NS = 16
NW = NC * NS

NP = 10240
RPT = NP // NS
CH = 64
EPW = E // NW
PADW = 240
IBK = 20
NBLK = (EPW + PADW) // (IBK * CH)
NB = 4

DCH = 128
NCH = (EPW + PADW) // DCH
NDS = 4

_MESH = plsc.VectorSubcoreMesh(
    core_axis_name="c", subcore_axis_name="s", num_cores=NC, num_subcores=NS
)


@functools.partial(
    pl.kernel,
    out_type=jax.ShapeDtypeStruct((NC, NP), jnp.float32),
    mesh=_MESH,
    scratch_types=[
        pltpu.VMEM_SHARED((NP,), jnp.float32),
        pltpu.VMEM((NCH, DCH), jnp.int32),
        pltpu.VMEM((DCH,), jnp.float32),
        pltpu.VMEM((RPT,), jnp.float32),
        [pltpu.SemaphoreType.DMA] * NDS,
    ],
)
def _deg_pass(dst_hbm, out_hbm, deg_sh, idx_v, ones_v, zero_v, dsem):
    c = lax.axis_index("c")
    s = lax.axis_index("s")
    w = c * NS + s

    def fill(i, _):
        zero_v[pl.ds(i * 16, 16)] = jnp.zeros((16,), jnp.float32)
        return ()

    lax.fori_loop(0, RPT // 16, fill, ())
    for j in range(DCH // 16):
        ones_v[pl.ds(j * 16, 16)] = jnp.ones((16,), jnp.float32)

    pltpu.sync_copy(dst_hbm.at[w], idx_v)
    pltpu.sync_copy(zero_v, deg_sh.at[pl.ds(s * RPT, RPT)])
    plsc.subcore_barrier()

    def body(t, _):
        for r in range(NDS):
            g = t * NDS + r

            @pl.when(t > 0)
            def _():
                pltpu.make_async_copy(
                    ones_v, deg_sh.at[idx_v.at[0]], dsem[r]).wait()

            pltpu.async_copy(ones_v, deg_sh.at[idx_v.at[g]], dsem[r],
                             add=True)
        return ()

    lax.fori_loop(0, NCH // NDS, body, ())
    for r in range(NDS):
        pltpu.make_async_copy(ones_v, deg_sh.at[idx_v.at[0]], dsem[r]).wait()
    plsc.subcore_barrier()
    pltpu.sync_copy(deg_sh.at[pl.ds(s * RPT, RPT)],
                    out_hbm.at[c, pl.ds(s * RPT, RPT)])


@functools.partial(
    pl.kernel,
    out_type=jax.ShapeDtypeStruct((NC, NP, D), jnp.float32),
    mesh=_MESH,
    scratch_types=[
        pltpu.VMEM_SHARED((NP, D), jnp.float32),
        [pltpu.VMEM((IBK, CH), jnp.int32)] * 2,
        [pltpu.VMEM((IBK, CH), jnp.int32)] * 2,
        [pltpu.VMEM((CH, D), jnp.float32)] * NB,
        pltpu.VMEM((16, D), jnp.float32),
        [pltpu.SemaphoreType.DMA] * NB,
        [pltpu.SemaphoreType.DMA] * NB,
        [pltpu.SemaphoreType.DMA] * 2,
    ],
)
def _edge_pass(hs_hbm, src_hbm, dst_hbm, out_hbm,
               acc_sh, srcb, dstb, rows, zv, gsem, ssem, isem):
    c = lax.axis_index("c")
    s = lax.axis_index("s")
    w = c * NS + s

    def fill(i, _):
        for j in range(D // 16):
            zv[i, pl.ds(j * 16, 16)] = jnp.zeros((16,), jnp.float32)
        return ()

    lax.fori_loop(0, 16, fill, ())

    def zacc(k, _):
        pltpu.sync_copy(zv, acc_sh.at[pl.ds(s * RPT + k * 16, 16)])
        return ()

    lax.fori_loop(0, RPT // 16, zacc, ())
    plsc.subcore_barrier()

    pltpu.sync_copy(src_hbm.at[w, 0], srcb[0])
    pltpu.sync_copy(dst_hbm.at[w, 0], dstb[0])

    def proc(sb, db):
        for b in range(NB - 1):
            pltpu.async_copy(hs_hbm.at[sb.at[b]], rows[b], gsem[b])

        def grp(t, _):
            scat = [None] * NB
            for b in range(NB):
                j = t * NB + b
                pltpu.make_async_copy(
                    hs_hbm.at[sb.at[j]], rows[b], gsem[b]).wait()
                scat[b] = pltpu.async_copy(
                    rows[b], acc_sh.at[db.at[j]], ssem[b], add=True)
                nb = (b + NB - 1) % NB
                if b == 0:
                    @pl.when(t > 0)
                    def _():
                        pltpu.make_async_copy(
                            rows[nb], acc_sh.at[db.at[0]], ssem[nb]).wait()

                    pltpu.async_copy(
                        hs_hbm.at[sb.at[j + NB - 1]], rows[nb], gsem[nb])
                else:
                    prev = scat[b - 1]

                    @pl.when(t < IBK // NB - 1)
                    def _():
                        prev.wait()
                        pltpu.async_copy(
                            hs_hbm.at[sb.at[j + NB - 1]], rows[nb],
                            gsem[nb])
            return ()

        lax.fori_loop(0, IBK // NB, grp, ())
        for b in range(NB):
            pltpu.make_async_copy(
                rows[b], acc_sh.at[db.at[0]], ssem[b]).wait()

    def pair(u, _):
        for pb in range(2):
            bk = 2 * u + pb
            nxt = pb ^ 1

            @pl.when(bk < NBLK - 1)
            def _():
                pltpu.async_copy(src_hbm.at[w, bk + 1], srcb[nxt], isem[0])
                pltpu.async_copy(dst_hbm.at[w, bk + 1], dstb[nxt], isem[1])

            proc(srcb[pb], dstb[pb])

            @pl.when(bk < NBLK - 1)
            def _():
                pltpu.make_async_copy(
                    src_hbm.at[w, 0], srcb[nxt], isem[0]).wait()
                pltpu.make_async_copy(
                    dst_hbm.at[w, 0], dstb[nxt], isem[1]).wait()
        return ()

    lax.fori_loop(0, NBLK // 2, pair, ())
    plsc.subcore_barrier()
    pltpu.sync_copy(acc_sh.at[pl.ds(s * RPT, RPT)],
                    out_hbm.at[c, pl.ds(s * RPT, RPT)])


RB = 512


def _dinv(d0, d1):
    return lax.rsqrt(d0 + d1 + 1.0)


def _prep_body(x_ref, w_ref, d0_ref, d1_ref, o_ref):
    dinv = _dinv(d0_ref[...], d1_ref[...])
    o_ref[...] = jnp.dot(x_ref[...], w_ref[...],
                         preferred_element_type=jnp.float32) * dinv


def _mid_body(a0_ref, a1_ref, hs_ref, d0_ref, d1_ref, b_ref, w_ref, o_ref):
    dinv = _dinv(d0_ref[...], d1_ref[...])
    t = (a0_ref[...] + a1_ref[...] + hs_ref[...]) * dinv + b_ref[...]
    x2 = jnp.maximum(t, 0.0)
    o_ref[...] = jnp.dot(x2, w_ref[...],
                         preferred_element_type=jnp.float32) * dinv


def _final_body(a0_ref, a1_ref, hs_ref, d0_ref, d1_ref, b_ref, o_ref):
    dinv = _dinv(d0_ref[...], d1_ref[...])
    t = (a0_ref[...] + a1_ref[...] + hs_ref[...]) * dinv + b_ref[...]
    o_ref[...] = jnp.maximum(t, 0.0)


_row_spec = pl.BlockSpec((RB, D), lambda i: (i, 0))
_col_spec = pl.BlockSpec((RB, 1), lambda i: (i, 0))
_mat_spec = pl.BlockSpec((D, D), lambda i: (0, 0))
_bias_spec = pl.BlockSpec((1, D), lambda i: (0, 0))
_out_row = jax.ShapeDtypeStruct((NP, D), jnp.float32)

_prep = pl.pallas_call(
    _prep_body,
    grid=(NP // RB,),
    in_specs=[_row_spec, _mat_spec, _col_spec, _col_spec],
    out_specs=_row_spec,
    out_shape=_out_row,
)

_mid = pl.pallas_call(
    _mid_body,
    grid=(NP // RB,),
    in_specs=[_row_spec, _row_spec, _row_spec, _col_spec, _col_spec,
              _bias_spec, _mat_spec],
    out_specs=_row_spec,
    out_shape=_out_row,
)

_final = pl.pallas_call(
    _final_body,
    grid=(NP // RB,),
    in_specs=[_row_spec, _row_spec, _row_spec, _col_spec, _col_spec,
              _bias_spec],
    out_specs=_row_spec,
    out_shape=_out_row,
)

_PAD_SRC = np.asarray(
    (np.arange(NW * PADW, dtype=np.int64) * 41) % N, np.int32
).reshape(NW, PADW)
_PAD_DST = np.asarray(
    N + (np.arange(NW * PADW, dtype=np.int64) * 7) % (NP - N), np.int32
).reshape(NW, PADW)


def kernel(x, edge_index, W1, b1, W2, b2):
    src_p = jnp.concatenate(
        [edge_index[0].reshape(NW, EPW), jnp.asarray(_PAD_SRC)], axis=1)
    dst_p = jnp.concatenate(
        [edge_index[1].reshape(NW, EPW), jnp.asarray(_PAD_DST)], axis=1)
    src_r = src_p.reshape(NW, NBLK, IBK, CH)
    dst_r = dst_p.reshape(NW, NBLK, IBK, CH)
    dst_deg = dst_p.reshape(NW, NCH, DCH)
    x_pad = jnp.concatenate(
        [x, jnp.zeros((NP - N, D), jnp.float32)], axis=0)

    degp = _deg_pass(dst_deg)
    d0 = degp[0].reshape(NP, 1)
    d1 = degp[1].reshape(NP, 1)

    hs1 = _prep(x_pad, W1, d0, d1)
    acc1 = _edge_pass(hs1, src_r, dst_r)
    hs2 = _mid(acc1[0], acc1[1], hs1, d0, d1, b1.reshape(1, D), W2)
    acc2 = _edge_pass(hs2, src_r, dst_r)
    out = _final(acc2[0], acc2[1], hs2, d0, d1, b2.reshape(1, D))
    return out[:N]


# speedup vs baseline: 1.0967x; 1.0464x over previous
import functools

import jax
import jax.numpy as jnp
import numpy as np
from jax import lax
from jax.experimental import pallas as pl
from jax.experimental.pallas import tpu as pltpu
from jax.experimental.pallas import tpu_sc as plsc

N = 10000
E = 320000
D = 128

NC = 2
NS = 16
NW = NC * NS

NP = 10240
RPT = NP // NS
CH = 64
EPW = E // NW
PADW = 240
IBK = 20
NBLK = (EPW + PADW) // (IBK * CH)
NB = 4

DCH = 128
NCH = (EPW + PADW) // DCH
NDS = 4

_MESH = plsc.VectorSubcoreMesh(
    core_axis_name="c", subcore_axis_name="s", num_cores=NC, num_subcores=NS
)


@functools.partial(
    pl.kernel,
    out_type=jax.ShapeDtypeStruct((NC, NP), jnp.float32),
    mesh=_MESH,
    scratch_types=[
        pltpu.VMEM_SHARED((NP,), jnp.float32),
        pltpu.VMEM((NCH, DCH), jnp.int32),
        pltpu.VMEM((DCH,), jnp.float32),
        pltpu.VMEM((RPT,), jnp.float32),
        [pltpu.SemaphoreType.DMA] * NDS,
    ],
)
def _deg_pass(dst_hbm, out_hbm, deg_sh, idx_v, ones_v, zero_v, dsem):
    c = lax.axis_index("c")
    s = lax.axis_index("s")
    w = c * NS + s

    def fill(i, _):
        zero_v[pl.ds(i * 16, 16)] = jnp.zeros((16,), jnp.float32)
        return ()

    lax.fori_loop(0, RPT // 16, fill, ())
    for j in range(DCH // 16):
        ones_v[pl.ds(j * 16, 16)] = jnp.ones((16,), jnp.float32)

    pltpu.sync_copy(dst_hbm.at[w], idx_v)
    pltpu.sync_copy(zero_v, deg_sh.at[pl.ds(s * RPT, RPT)])
    plsc.subcore_barrier()

    def body(t, _):
        for r in range(NDS):
            g = t * NDS + r

            @pl.when(t > 0)
            def _():
                pltpu.make_async_copy(
                    ones_v, deg_sh.at[idx_v.at[0]], dsem[r]).wait()

            pltpu.async_copy(ones_v, deg_sh.at[idx_v.at[g]], dsem[r],
                             add=True)
        return ()

    lax.fori_loop(0, NCH // NDS, body, ())
    for r in range(NDS):
        pltpu.make_async_copy(ones_v, deg_sh.at[idx_v.at[0]], dsem[r]).wait()
    plsc.subcore_barrier()
    pltpu.sync_copy(deg_sh.at[pl.ds(s * RPT, RPT)],
                    out_hbm.at[c, pl.ds(s * RPT, RPT)])


@functools.partial(
    pl.kernel,
    out_type=jax.ShapeDtypeStruct((NC, NP, D), jnp.float32),
    mesh=_MESH,
    scratch_types=[
        pltpu.VMEM_SHARED((NP, D), jnp.float32),
        [pltpu.VMEM((IBK, CH), jnp.int32)] * 2,
        [pltpu.VMEM((IBK, CH), jnp.int32)] * 2,
        [pltpu.VMEM((CH, D), jnp.float32)] * NB,
        pltpu.VMEM((16, D), jnp.float32),
        [pltpu.SemaphoreType.DMA] * NB,
        [pltpu.SemaphoreType.DMA] * NB,
        [pltpu.SemaphoreType.DMA] * 2,
    ],
)
def _edge_pass(hs_hbm, src_hbm, dst_hbm, out_hbm,
               acc_sh, srcb, dstb, rows, zv, gsem, ssem, isem):
    c = lax.axis_index("c")
    s = lax.axis_index("s")
    w = c * NS + s

    def fill(i, _):
        for j in range(D // 16):
            zv[i, pl.ds(j * 16, 16)] = jnp.zeros((16,), jnp.float32)
        return ()

    lax.fori_loop(0, 16, fill, ())

    def zacc(k, _):
        pltpu.sync_copy(zv, acc_sh.at[pl.ds(s * RPT + k * 16, 16)])
        return ()

    lax.fori_loop(0, RPT // 16, zacc, ())
    plsc.subcore_barrier()

    pltpu.sync_copy(src_hbm.at[w, 0], srcb[0])
    pltpu.sync_copy(dst_hbm.at[w, 0], dstb[0])

    def proc(sb, db):
        for b in range(NB - 1):
            pltpu.async_copy(hs_hbm.at[sb.at[b]], rows[b], gsem[b])

        def grp(t, _):
            scat = [None] * NB
            for b in range(NB):
                j = t * NB + b
                pltpu.make_async_copy(
                    hs_hbm.at[sb.at[j]], rows[b], gsem[b]).wait()
                scat[b] = pltpu.async_copy(
                    rows[b], acc_sh.at[db.at[j]], ssem[b], add=True)
                nb = (b + NB - 1) % NB
                if b == 0:
                    @pl.when(t > 0)
                    def _():
                        pltpu.make_async_copy(
                            rows[nb], acc_sh.at[db.at[0]], ssem[nb]).wait()

                    pltpu.async_copy(
                        hs_hbm.at[sb.at[j + NB - 1]], rows[nb], gsem[nb])
                else:
                    prev = scat[b - 1]

                    @pl.when(t < IBK // NB - 1)
                    def _():
                        prev.wait()
                        pltpu.async_copy(
                            hs_hbm.at[sb.at[j + NB - 1]], rows[nb],
                            gsem[nb])
            return ()

        lax.fori_loop(0, IBK // NB, grp, ())
        for b in range(NB):
            pltpu.make_async_copy(
                rows[b], acc_sh.at[db.at[0]], ssem[b]).wait()

    def pair(u, _):
        for pb in range(2):
            bk = 2 * u + pb
            nxt = pb ^ 1

            @pl.when(bk < NBLK - 1)
            def _():
                pltpu.async_copy(src_hbm.at[w, bk + 1], srcb[nxt], isem[0])
                pltpu.async_copy(dst_hbm.at[w, bk + 1], dstb[nxt], isem[1])

            proc(srcb[pb], dstb[pb])

            @pl.when(bk < NBLK - 1)
            def _():
                pltpu.make_async_copy(
                    src_hbm.at[w, 0], srcb[nxt], isem[0]).wait()
                pltpu.make_async_copy(
                    dst_hbm.at[w, 0], dstb[nxt], isem[1]).wait()
        return ()

    lax.fori_loop(0, NBLK // 2, pair, ())
    plsc.subcore_barrier()
    pltpu.sync_copy(acc_sh.at[pl.ds(s * RPT, RPT)],
                    out_hbm.at[c, pl.ds(s * RPT, RPT)])


RB = 1024


def _dinv(d0, d1):
    return lax.rsqrt(d0 + d1 + 1.0)


def _prep_body(x_ref, w_ref, d0_ref, d1_ref, o_ref):
    dinv = _dinv(d0_ref[...], d1_ref[...])
    o_ref[...] = jnp.dot(x_ref[...], w_ref[...],
                         preferred_element_type=jnp.float32) * dinv


def _mid_body(a0_ref, a1_ref, hs_ref, d0_ref, d1_ref, b_ref, w_ref, o_ref):
    dinv = _dinv(d0_ref[...], d1_ref[...])
    t = (a0_ref[...] + a1_ref[...] + hs_ref[...]) * dinv + b_ref[...]
    x2 = jnp.maximum(t, 0.0)
    o_ref[...] = jnp.dot(x2, w_ref[...],
                         preferred_element_type=jnp.float32) * dinv


def _final_body(a0_ref, a1_ref, hs_ref, d0_ref, d1_ref, b_ref, o_ref):
    dinv = _dinv(d0_ref[...], d1_ref[...])
    t = (a0_ref[...] + a1_ref[...] + hs_ref[...]) * dinv + b_ref[...]
    o_ref[...] = jnp.maximum(t, 0.0)


_row_spec = pl.BlockSpec((RB, D), lambda i: (i, 0))
_col_spec = pl.BlockSpec((RB, 1), lambda i: (i, 0))
_mat_spec = pl.BlockSpec((D, D), lambda i: (0, 0))
_bias_spec = pl.BlockSpec((1, D), lambda i: (0, 0))
_out_row = jax.ShapeDtypeStruct((NP, D), jnp.float32)

_prep = pl.pallas_call(
    _prep_body,
    grid=(NP // RB,),
    in_specs=[_row_spec, _mat_spec, _col_spec, _col_spec],
    out_specs=_row_spec,
    out_shape=_out_row,
)

_mid = pl.pallas_call(
    _mid_body,
    grid=(NP // RB,),
    in_specs=[_row_spec, _row_spec, _row_spec, _col_spec, _col_spec,
              _bias_spec, _mat_spec],
    out_specs=_row_spec,
    out_shape=_out_row,
)

_final = pl.pallas_call(
    _final_body,
    grid=(NP // RB,),
    in_specs=[_row_spec, _row_spec, _row_spec, _col_spec, _col_spec,
              _bias_spec],
    out_specs=_row_spec,
    out_shape=_out_row,
)

_PAD_SRC = np.asarray(
    (np.arange(NW * PADW, dtype=np.int64) * 41) % N, np.int32
).reshape(NW, PADW)
_PAD_DST = np.asarray(
    N + (np.arange(NW * PADW, dtype=np.int64) * 7) % (NP - N), np.int32
).reshape(NW, PADW)


def kernel(x, edge_index, W1, b1, W2, b2):
    src_p = jnp.concatenate(
        [edge_index[0].reshape(NW, EPW), jnp.asarray(_PAD_SRC)], axis=1)
    dst_p = jnp.concatenate(
        [edge_index[1].reshape(NW, EPW), jnp.asarray(_PAD_DST)], axis=1)
    src_r = src_p.reshape(NW, NBLK, IBK, CH)
    dst_r = dst_p.reshape(NW, NBLK, IBK, CH)
    dst_deg = dst_p.reshape(NW, NCH, DCH)
    x_pad = jnp.concatenate(
        [x, jnp.zeros((NP - N, D), jnp.float32)], axis=0)

    degp = _deg_pass(dst_deg)
    d0 = degp[0].reshape(NP, 1)
    d1 = degp[1].reshape(NP, 1)

    hs1 = _prep(x_pad, W1, d0, d1)
    acc1 = _edge_pass(hs1, src_r, dst_r)
    hs2 = _mid(acc1[0], acc1[1], hs1, d0, d1, b1.reshape(1, D), W2)
    acc2 = _edge_pass(hs2, src_r, dst_r)
    out = _final(acc2[0], acc2[1], hs2, d0, d1, b2.reshape(1, D))
    return out[:N]


# speedup vs baseline: 1.1156x; 1.0172x over previous
import functools

import jax
import jax.numpy as jnp
import numpy as np
from jax import lax
from jax.experimental import pallas as pl
from jax.experimental.pallas import tpu as pltpu
from jax.experimental.pallas import tpu_sc as plsc

N = 10000
E = 320000
D = 128

NC = 2
NS = 16
NW = NC * NS

NP = 10240
RPT = NP // NS
CH = 64
EPW = E // NW
PADW = 240
IBK = 20
NBLK = (EPW + PADW) // (IBK * CH)
NB = 4

DCH = 128
NCH = (EPW + PADW) // DCH
NDS = 4

_MESH = plsc.VectorSubcoreMesh(
    core_axis_name="c", subcore_axis_name="s", num_cores=NC, num_subcores=NS
)


@functools.partial(
    pl.kernel,
    out_type=jax.ShapeDtypeStruct((NC, NP), jnp.float32),
    mesh=_MESH,
    scratch_types=[
        pltpu.VMEM_SHARED((NP,), jnp.float32),
        pltpu.VMEM((NCH, DCH), jnp.int32),
        pltpu.VMEM((DCH,), jnp.float32),
        pltpu.VMEM((RPT,), jnp.float32),
        [pltpu.SemaphoreType.DMA] * NDS,
    ],
)
def _deg_pass(dst_hbm, out_hbm, deg_sh, idx_v, ones_v, zero_v, dsem):
    c = lax.axis_index("c")
    s = lax.axis_index("s")
    w = c * NS + s

    def fill(i, _):
        zero_v[pl.ds(i * 16, 16)] = jnp.zeros((16,), jnp.float32)
        return ()

    lax.fori_loop(0, RPT // 16, fill, ())
    for j in range(DCH // 16):
        ones_v[pl.ds(j * 16, 16)] = jnp.ones((16,), jnp.float32)

    pltpu.sync_copy(dst_hbm.at[w], idx_v)
    pltpu.sync_copy(zero_v, deg_sh.at[pl.ds(s * RPT, RPT)])
    plsc.subcore_barrier()

    def body(t, _):
        for r in range(NDS):
            g = t * NDS + r

            @pl.when(t > 0)
            def _():
                pltpu.make_async_copy(
                    ones_v, deg_sh.at[idx_v.at[0]], dsem[r]).wait()

            pltpu.async_copy(ones_v, deg_sh.at[idx_v.at[g]], dsem[r],
                             add=True)
        return ()

    lax.fori_loop(0, NCH // NDS, body, ())
    for r in range(NDS):
        pltpu.make_async_copy(ones_v, deg_sh.at[idx_v.at[0]], dsem[r]).wait()
    plsc.subcore_barrier()
    pltpu.sync_copy(deg_sh.at[pl.ds(s * RPT, RPT)],
                    out_hbm.at[c, pl.ds(s * RPT, RPT)])


@functools.partial(
    pl.kernel,
    out_type=jax.ShapeDtypeStruct((NC, NP, D), jnp.float32),
    mesh=_MESH,
    scratch_types=[
        pltpu.VMEM_SHARED((NP, D), jnp.float32),
        [pltpu.VMEM((IBK, CH), jnp.int32)] * 2,
        [pltpu.VMEM((IBK, CH), jnp.int32)] * 2,
        [pltpu.VMEM((CH, D), jnp.float32)] * NB,
        pltpu.VMEM((16, D), jnp.float32),
        [pltpu.SemaphoreType.DMA] * NB,
        [pltpu.SemaphoreType.DMA] * NB,
        [pltpu.SemaphoreType.DMA] * 2,
    ],
)
def _edge_pass(hs_hbm, src_hbm, dst_hbm, out_hbm,
               acc_sh, srcb, dstb, rows, zv, gsem, ssem, isem):
    c = lax.axis_index("c")
    s = lax.axis_index("s")
    w = c * NS + s

    def fill(i, _):
        for j in range(D // 16):
            zv[i, pl.ds(j * 16, 16)] = jnp.zeros((16,), jnp.float32)
        return ()

    lax.fori_loop(0, 16, fill, ())

    def zacc(k, _):
        pltpu.sync_copy(zv, acc_sh.at[pl.ds(s * RPT + k * 16, 16)])
        return ()

    lax.fori_loop(0, RPT // 16, zacc, ())
    plsc.subcore_barrier()

    pltpu.sync_copy(src_hbm.at[w, 0], srcb[0])
    pltpu.sync_copy(dst_hbm.at[w, 0], dstb[0])

    def proc(sb, db):
        for b in range(NB - 1):
            pltpu.async_copy(hs_hbm.at[sb.at[b]], rows[b], gsem[b])

        def grp(t, _):
            scat = [None] * NB
            for b in range(NB):
                j = t * NB + b
                pltpu.make_async_copy(
                    hs_hbm.at[sb.at[j]], rows[b], gsem[b]).wait()
                scat[b] = pltpu.async_copy(
                    rows[b], acc_sh.at[db.at[j]], ssem[b], add=True)
                nb = (b + NB - 1) % NB
                if b == 0:
                    @pl.when(t > 0)
                    def _():
                        pltpu.make_async_copy(
                            rows[nb], acc_sh.at[db.at[0]], ssem[nb]).wait()

                    pltpu.async_copy(
                        hs_hbm.at[sb.at[j + NB - 1]], rows[nb], gsem[nb])
                else:
                    prev = scat[b - 1]

                    @pl.when(t < IBK // NB - 1)
                    def _():
                        prev.wait()
                        pltpu.async_copy(
                            hs_hbm.at[sb.at[j + NB - 1]], rows[nb],
                            gsem[nb])
            return ()

        lax.fori_loop(0, IBK // NB, grp, ())
        for b in range(NB):
            pltpu.make_async_copy(
                rows[b], acc_sh.at[db.at[0]], ssem[b]).wait()

    def pair(u, _):
        for pb in range(2):
            bk = 2 * u + pb
            nxt = pb ^ 1

            @pl.when(bk < NBLK - 1)
            def _():
                pltpu.async_copy(src_hbm.at[w, bk + 1], srcb[nxt], isem[0])
                pltpu.async_copy(dst_hbm.at[w, bk + 1], dstb[nxt], isem[1])

            proc(srcb[pb], dstb[pb])

            @pl.when(bk < NBLK - 1)
            def _():
                pltpu.make_async_copy(
                    src_hbm.at[w, 0], srcb[nxt], isem[0]).wait()
                pltpu.make_async_copy(
                    dst_hbm.at[w, 0], dstb[nxt], isem[1]).wait()
        return ()

    lax.fori_loop(0, NBLK // 2, pair, ())
    plsc.subcore_barrier()
    pltpu.sync_copy(acc_sh.at[pl.ds(s * RPT, RPT)],
                    out_hbm.at[c, pl.ds(s * RPT, RPT)])


RB = 2048


def _dinv(d0, d1):
    return lax.rsqrt(d0 + d1 + 1.0)


def _prep_body(x_ref, w_ref, d0_ref, d1_ref, o_ref):
    dinv = _dinv(d0_ref[...], d1_ref[...])
    o_ref[...] = jnp.dot(x_ref[...], w_ref[...],
                         preferred_element_type=jnp.float32) * dinv


def _mid_body(a0_ref, a1_ref, hs_ref, d0_ref, d1_ref, b_ref, w_ref, o_ref):
    dinv = _dinv(d0_ref[...], d1_ref[...])
    t = (a0_ref[...] + a1_ref[...] + hs_ref[...]) * dinv + b_ref[...]
    x2 = jnp.maximum(t, 0.0)
    o_ref[...] = jnp.dot(x2, w_ref[...],
                         preferred_element_type=jnp.float32) * dinv


def _final_body(a0_ref, a1_ref, hs_ref, d0_ref, d1_ref, b_ref, o_ref):
    dinv = _dinv(d0_ref[...], d1_ref[...])
    t = (a0_ref[...] + a1_ref[...] + hs_ref[...]) * dinv + b_ref[...]
    o_ref[...] = jnp.maximum(t, 0.0)


_row_spec = pl.BlockSpec((RB, D), lambda i: (i, 0))
_col_spec = pl.BlockSpec((RB, 1), lambda i: (i, 0))
_mat_spec = pl.BlockSpec((D, D), lambda i: (0, 0))
_bias_spec = pl.BlockSpec((1, D), lambda i: (0, 0))
_out_row = jax.ShapeDtypeStruct((NP, D), jnp.float32)

_prep = pl.pallas_call(
    _prep_body,
    grid=(NP // RB,),
    in_specs=[_row_spec, _mat_spec, _col_spec, _col_spec],
    out_specs=_row_spec,
    out_shape=_out_row,
)

_mid = pl.pallas_call(
    _mid_body,
    grid=(NP // RB,),
    in_specs=[_row_spec, _row_spec, _row_spec, _col_spec, _col_spec,
              _bias_spec, _mat_spec],
    out_specs=_row_spec,
    out_shape=_out_row,
)

_final = pl.pallas_call(
    _final_body,
    grid=(NP // RB,),
    in_specs=[_row_spec, _row_spec, _row_spec, _col_spec, _col_spec,
              _bias_spec],
    out_specs=_row_spec,
    out_shape=_out_row,
)

_PAD_SRC = np.asarray(
    (np.arange(NW * PADW, dtype=np.int64) * 41) % N, np.int32
).reshape(NW, PADW)
_PAD_DST = np.asarray(
    N + (np.arange(NW * PADW, dtype=np.int64) * 7) % (NP - N), np.int32
).reshape(NW, PADW)


def kernel(x, edge_index, W1, b1, W2, b2):
    src_p = jnp.concatenate(
        [edge_index[0].reshape(NW, EPW), jnp.asarray(_PAD_SRC)], axis=1)
    dst_p = jnp.concatenate(
        [edge_index[1].reshape(NW, EPW), jnp.asarray(_PAD_DST)], axis=1)
    src_r = src_p.reshape(NW, NBLK, IBK, CH)
    dst_r = dst_p.reshape(NW, NBLK, IBK, CH)
    dst_deg = dst_p.reshape(NW, NCH, DCH)
    x_pad = jnp.concatenate(
        [x, jnp.zeros((NP - N, D), jnp.float32)], axis=0)

    degp = _deg_pass(dst_deg)
    d0 = degp[0].reshape(NP, 1)
    d1 = degp[1].reshape(NP, 1)

    hs1 = _prep(x_pad, W1, d0, d1)
    acc1 = _edge_pass(hs1, src_r, dst_r)
    hs2 = _mid(acc1[0], acc1[1], hs1, d0, d1, b1.reshape(1, D), W2)
    acc2 = _edge_pass(hs2, src_r, dst_r)
    out = _final(acc2[0], acc2[1], hs2, d0, d1, b2.reshape(1, D))
    return out[:N]


# speedup vs baseline: 1.1172x; 1.0014x over previous
import functools

import jax
import jax.numpy as jnp
import numpy as np
from jax import lax
from jax.experimental import pallas as pl
from jax.experimental.pallas import tpu as pltpu
from jax.experimental.pallas import tpu_sc as plsc

N = 10000
E = 320000
D = 128

NC = 2
NS = 16
NW = NC * NS

NP = 10240
RPT = NP // NS
CH = 64
EPW = E // NW
PADW = 240
IBK = 20
NBLK = (EPW + PADW) // (IBK * CH)
NB = 4

DCH = 128
NCH = (EPW + PADW) // DCH
NDS = 4

_MESH = plsc.VectorSubcoreMesh(
    core_axis_name="c", subcore_axis_name="s", num_cores=NC, num_subcores=NS
)


@functools.partial(
    pl.kernel,
    out_type=jax.ShapeDtypeStruct((NC, NP), jnp.float32),
    mesh=_MESH,
    scratch_types=[
        pltpu.VMEM_SHARED((NP,), jnp.float32),
        pltpu.VMEM((NCH, DCH), jnp.int32),
        pltpu.VMEM((DCH,), jnp.float32),
        pltpu.VMEM((RPT,), jnp.float32),
        [pltpu.SemaphoreType.DMA] * NDS,
    ],
)
def _deg_pass(dst_hbm, out_hbm, deg_sh, idx_v, ones_v, zero_v, dsem):
    c = lax.axis_index("c")
    s = lax.axis_index("s")
    w = c * NS + s

    def fill(i, _):
        zero_v[pl.ds(i * 16, 16)] = jnp.zeros((16,), jnp.float32)
        return ()

    lax.fori_loop(0, RPT // 16, fill, ())
    for j in range(DCH // 16):
        ones_v[pl.ds(j * 16, 16)] = jnp.ones((16,), jnp.float32)

    pltpu.sync_copy(dst_hbm.at[w], idx_v)
    pltpu.sync_copy(zero_v, deg_sh.at[pl.ds(s * RPT, RPT)])
    plsc.subcore_barrier()

    def body(t, _):
        for r in range(NDS):
            g = t * NDS + r

            @pl.when(t > 0)
            def _():
                pltpu.make_async_copy(
                    ones_v, deg_sh.at[idx_v.at[0]], dsem[r]).wait()

            pltpu.async_copy(ones_v, deg_sh.at[idx_v.at[g]], dsem[r],
                             add=True)
        return ()

    lax.fori_loop(0, NCH // NDS, body, ())
    for r in range(NDS):
        pltpu.make_async_copy(ones_v, deg_sh.at[idx_v.at[0]], dsem[r]).wait()
    plsc.subcore_barrier()
    pltpu.sync_copy(deg_sh.at[pl.ds(s * RPT, RPT)],
                    out_hbm.at[c, pl.ds(s * RPT, RPT)])


@functools.partial(
    pl.kernel,
    out_type=jax.ShapeDtypeStruct((NC, NP, D), jnp.float32),
    mesh=_MESH,
    scratch_types=[
        pltpu.VMEM_SHARED((NP, D), jnp.float32),
        [pltpu.VMEM((IBK, CH), jnp.int32)] * 2,
        [pltpu.VMEM((IBK, CH), jnp.int32)] * 2,
        [pltpu.VMEM((CH, D), jnp.float32)] * NB,
        pltpu.VMEM((16, D), jnp.float32),
        [pltpu.SemaphoreType.DMA] * NB,
        [pltpu.SemaphoreType.DMA] * NB,
        [pltpu.SemaphoreType.DMA] * 2,
    ],
)
def _edge_pass(hs_hbm, src_hbm, dst_hbm, out_hbm,
               acc_sh, srcb, dstb, rows, zv, gsem, ssem, isem):
    c = lax.axis_index("c")
    s = lax.axis_index("s")
    w = c * NS + s

    def fill(i, _):
        for j in range(D // 16):
            zv[i, pl.ds(j * 16, 16)] = jnp.zeros((16,), jnp.float32)
        return ()

    lax.fori_loop(0, 16, fill, ())

    def zacc(k, _):
        pltpu.sync_copy(zv, acc_sh.at[pl.ds(s * RPT + k * 16, 16)])
        return ()

    lax.fori_loop(0, RPT // 16, zacc, ())
    plsc.subcore_barrier()

    pltpu.sync_copy(src_hbm.at[w, 0], srcb[0])
    pltpu.sync_copy(dst_hbm.at[w, 0], dstb[0])

    def proc(sb, db):
        for b in range(NB - 1):
            pltpu.async_copy(hs_hbm.at[sb.at[b]], rows[b], gsem[b])

        def grp(t, _):
            scat = [None] * NB
            for b in range(NB):
                j = t * NB + b
                pltpu.make_async_copy(
                    hs_hbm.at[sb.at[j]], rows[b], gsem[b]).wait()
                scat[b] = pltpu.async_copy(
                    rows[b], acc_sh.at[db.at[j]], ssem[b], add=True)
                nb = (b + NB - 1) % NB
                if b == 0:
                    @pl.when(t > 0)
                    def _():
                        pltpu.make_async_copy(
                            rows[nb], acc_sh.at[db.at[0]], ssem[nb]).wait()

                    pltpu.async_copy(
                        hs_hbm.at[sb.at[j + NB - 1]], rows[nb], gsem[nb])
                else:
                    prev = scat[b - 1]

                    @pl.when(t < IBK // NB - 1)
                    def _():
                        prev.wait()
                        pltpu.async_copy(
                            hs_hbm.at[sb.at[j + NB - 1]], rows[nb],
                            gsem[nb])
            return ()

        lax.fori_loop(0, IBK // NB, grp, ())
        for b in range(NB):
            pltpu.make_async_copy(
                rows[b], acc_sh.at[db.at[0]], ssem[b]).wait()

    def pair(u, _):
        for pb in range(2):
            bk = 2 * u + pb
            nxt = pb ^ 1

            @pl.when(bk < NBLK - 1)
            def _():
                pltpu.async_copy(src_hbm.at[w, bk + 1], srcb[nxt], isem[0])
                pltpu.async_copy(dst_hbm.at[w, bk + 1], dstb[nxt], isem[1])

            proc(srcb[pb], dstb[pb])

            @pl.when(bk < NBLK - 1)
            def _():
                pltpu.make_async_copy(
                    src_hbm.at[w, 0], srcb[nxt], isem[0]).wait()
                pltpu.make_async_copy(
                    dst_hbm.at[w, 0], dstb[nxt], isem[1]).wait()
        return ()

    lax.fori_loop(0, NBLK // 2, pair, ())
    plsc.subcore_barrier()
    pltpu.sync_copy(acc_sh.at[pl.ds(s * RPT, RPT)],
                    out_hbm.at[c, pl.ds(s * RPT, RPT)])


RB = 2560


def _dinv(d0, d1):
    return lax.rsqrt(d0 + d1 + 1.0)


def _prep_body(x_ref, w_ref, d0_ref, d1_ref, o_ref):
    dinv = _dinv(d0_ref[...], d1_ref[...])
    o_ref[...] = jnp.dot(x_ref[...], w_ref[...],
                         preferred_element_type=jnp.float32) * dinv


def _mid_body(a0_ref, a1_ref, hs_ref, d0_ref, d1_ref, b_ref, w_ref, o_ref):
    dinv = _dinv(d0_ref[...], d1_ref[...])
    t = (a0_ref[...] + a1_ref[...] + hs_ref[...]) * dinv + b_ref[...]
    x2 = jnp.maximum(t, 0.0)
    o_ref[...] = jnp.dot(x2, w_ref[...],
                         preferred_element_type=jnp.float32) * dinv


def _final_body(a0_ref, a1_ref, hs_ref, d0_ref, d1_ref, b_ref, o_ref):
    dinv = _dinv(d0_ref[...], d1_ref[...])
    t = (a0_ref[...] + a1_ref[...] + hs_ref[...]) * dinv + b_ref[...]
    o_ref[...] = jnp.maximum(t, 0.0)


_row_spec = pl.BlockSpec((RB, D), lambda i: (i, 0))
_col_spec = pl.BlockSpec((RB, 1), lambda i: (i, 0))
_mat_spec = pl.BlockSpec((D, D), lambda i: (0, 0))
_bias_spec = pl.BlockSpec((1, D), lambda i: (0, 0))
_out_row = jax.ShapeDtypeStruct((NP, D), jnp.float32)

_prep = pl.pallas_call(
    _prep_body,
    grid=(NP // RB,),
    in_specs=[_row_spec, _mat_spec, _col_spec, _col_spec],
    out_specs=_row_spec,
    out_shape=_out_row,
)

_mid = pl.pallas_call(
    _mid_body,
    grid=(NP // RB,),
    in_specs=[_row_spec, _row_spec, _row_spec, _col_spec, _col_spec,
              _bias_spec, _mat_spec],
    out_specs=_row_spec,
    out_shape=_out_row,
)

_final = pl.pallas_call(
    _final_body,
    grid=(NP // RB,),
    in_specs=[_row_spec, _row_spec, _row_spec, _col_spec, _col_spec,
              _bias_spec],
    out_specs=_row_spec,
    out_shape=_out_row,
)

_PAD_SRC = np.asarray(
    (np.arange(NW * PADW, dtype=np.int64) * 41) % N, np.int32
).reshape(NW, PADW)
_PAD_DST = np.asarray(
    N + (np.arange(NW * PADW, dtype=np.int64) * 7) % (NP - N), np.int32
).reshape(NW, PADW)


def kernel(x, edge_index, W1, b1, W2, b2):
    src_p = jnp.concatenate(
        [edge_index[0].reshape(NW, EPW), jnp.asarray(_PAD_SRC)], axis=1)
    dst_p = jnp.concatenate(
        [edge_index[1].reshape(NW, EPW), jnp.asarray(_PAD_DST)], axis=1)
    src_r = src_p.reshape(NW, NBLK, IBK, CH)
    dst_r = dst_p.reshape(NW, NBLK, IBK, CH)
    dst_deg = dst_p.reshape(NW, NCH, DCH)
    x_pad = jnp.concatenate(
        [x, jnp.zeros((NP - N, D), jnp.float32)], axis=0)

    degp = _deg_pass(dst_deg)
    d0 = degp[0].reshape(NP, 1)
    d1 = degp[1].reshape(NP, 1)

    hs1 = _prep(x_pad, W1, d0, d1)
    acc1 = _edge_pass(hs1, src_r, dst_r)
    hs2 = _mid(acc1[0], acc1[1], hs1, d0, d1, b1.reshape(1, D), W2)
    acc2 = _edge_pass(hs2, src_r, dst_r)
    out = _final(acc2[0], acc2[1], hs2, d0, d1, b2.reshape(1, D))
    return out[:N]


# speedup vs baseline: 1.1295x; 1.0111x over previous
import functools

import jax
import jax.numpy as jnp
import numpy as np
from jax import lax
from jax.experimental import pallas as pl
from jax.experimental.pallas import tpu as pltpu
from jax.experimental.pallas import tpu_sc as plsc

N = 10000
E = 320000
D = 128

NC = 2
NS = 16
NW = NC * NS

NP = 10240
RPT = NP // NS
CH = 64
EPW = E // NW
PADW = 240
IBK = 20
NBLK = (EPW + PADW) // (IBK * CH)
NB = 4

DCH = 128
NCH = (EPW + PADW) // DCH
NDS = 4

_MESH = plsc.VectorSubcoreMesh(
    core_axis_name="c", subcore_axis_name="s", num_cores=NC, num_subcores=NS
)


@functools.partial(
    pl.kernel,
    out_type=jax.ShapeDtypeStruct((NC, NP), jnp.float32),
    mesh=_MESH,
    scratch_types=[
        pltpu.VMEM_SHARED((NP,), jnp.float32),
        pltpu.VMEM((NCH, DCH), jnp.int32),
        pltpu.VMEM((DCH,), jnp.float32),
        pltpu.VMEM((RPT,), jnp.float32),
        [pltpu.SemaphoreType.DMA] * NDS,
    ],
)
def _deg_pass(dst_hbm, out_hbm, deg_sh, idx_v, ones_v, zero_v, dsem):
    c = lax.axis_index("c")
    s = lax.axis_index("s")
    w = c * NS + s

    def fill(i, _):
        zero_v[pl.ds(i * 16, 16)] = jnp.zeros((16,), jnp.float32)
        return ()

    lax.fori_loop(0, RPT // 16, fill, ())
    for j in range(DCH // 16):
        ones_v[pl.ds(j * 16, 16)] = jnp.ones((16,), jnp.float32)

    pltpu.sync_copy(dst_hbm.at[w], idx_v)
    pltpu.sync_copy(zero_v, deg_sh.at[pl.ds(s * RPT, RPT)])
    plsc.subcore_barrier()

    def body(t, _):
        for r in range(NDS):
            g = t * NDS + r

            @pl.when(t > 0)
            def _():
                pltpu.make_async_copy(
                    ones_v, deg_sh.at[idx_v.at[0]], dsem[r]).wait()

            pltpu.async_copy(ones_v, deg_sh.at[idx_v.at[g]], dsem[r],
                             add=True)
        return ()

    lax.fori_loop(0, NCH // NDS, body, ())
    for r in range(NDS):
        pltpu.make_async_copy(ones_v, deg_sh.at[idx_v.at[0]], dsem[r]).wait()
    plsc.subcore_barrier()
    pltpu.sync_copy(deg_sh.at[pl.ds(s * RPT, RPT)],
                    out_hbm.at[c, pl.ds(s * RPT, RPT)])


@functools.partial(
    pl.kernel,
    out_type=jax.ShapeDtypeStruct((NC, NP, D), jnp.float32),
    mesh=_MESH,
    scratch_types=[
        pltpu.VMEM_SHARED((NP, D), jnp.float32),
        [pltpu.VMEM((IBK, CH), jnp.int32)] * 2,
        [pltpu.VMEM((IBK, CH), jnp.int32)] * 2,
        [pltpu.VMEM((CH, D), jnp.float32)] * NB,
        pltpu.VMEM((16, D), jnp.float32),
        [pltpu.SemaphoreType.DMA] * NB,
        [pltpu.SemaphoreType.DMA] * NB,
        [pltpu.SemaphoreType.DMA] * 2,
    ],
)
def _edge_pass(hs_hbm, src_hbm, dst_hbm, out_hbm,
               acc_sh, srcb, dstb, rows, zv, gsem, ssem, isem):
    c = lax.axis_index("c")
    s = lax.axis_index("s")
    w = c * NS + s

    def fill(i, _):
        for j in range(D // 16):
            zv[i, pl.ds(j * 16, 16)] = jnp.zeros((16,), jnp.float32)
        return ()

    lax.fori_loop(0, 16, fill, ())

    def zacc(t, _):
        for r in range(NB):
            k = t * NB + r

            @pl.when(t > 0)
            def _():
                pltpu.make_async_copy(
                    zv, acc_sh.at[pl.ds(s * RPT, 16)], gsem[r]).wait()

            pltpu.async_copy(
                zv, acc_sh.at[pl.ds(s * RPT + k * 16, 16)], gsem[r])
        return ()

    lax.fori_loop(0, RPT // 16 // NB, zacc, ())
    for r in range(NB):
        pltpu.make_async_copy(
            zv, acc_sh.at[pl.ds(s * RPT, 16)], gsem[r]).wait()
    plsc.subcore_barrier()

    pltpu.sync_copy(src_hbm.at[w, 0], srcb[0])
    pltpu.sync_copy(dst_hbm.at[w, 0], dstb[0])

    def proc(sb, db):
        for b in range(NB - 1):
            pltpu.async_copy(hs_hbm.at[sb.at[b]], rows[b], gsem[b])

        def grp(t, _):
            scat = [None] * NB
            for b in range(NB):
                j = t * NB + b
                pltpu.make_async_copy(
                    hs_hbm.at[sb.at[j]], rows[b], gsem[b]).wait()
                scat[b] = pltpu.async_copy(
                    rows[b], acc_sh.at[db.at[j]], ssem[b], add=True)
                nb = (b + NB - 1) % NB
                if b == 0:
                    @pl.when(t > 0)
                    def _():
                        pltpu.make_async_copy(
                            rows[nb], acc_sh.at[db.at[0]], ssem[nb]).wait()

                    pltpu.async_copy(
                        hs_hbm.at[sb.at[j + NB - 1]], rows[nb], gsem[nb])
                else:
                    prev = scat[b - 1]

                    @pl.when(t < IBK // NB - 1)
                    def _():
                        prev.wait()
                        pltpu.async_copy(
                            hs_hbm.at[sb.at[j + NB - 1]], rows[nb],
                            gsem[nb])
            return ()

        lax.fori_loop(0, IBK // NB, grp, ())
        for b in range(NB):
            pltpu.make_async_copy(
                rows[b], acc_sh.at[db.at[0]], ssem[b]).wait()

    def pair(u, _):
        for pb in range(2):
            bk = 2 * u + pb
            nxt = pb ^ 1

            @pl.when(bk < NBLK - 1)
            def _():
                pltpu.async_copy(src_hbm.at[w, bk + 1], srcb[nxt], isem[0])
                pltpu.async_copy(dst_hbm.at[w, bk + 1], dstb[nxt], isem[1])

            proc(srcb[pb], dstb[pb])

            @pl.when(bk < NBLK - 1)
            def _():
                pltpu.make_async_copy(
                    src_hbm.at[w, 0], srcb[nxt], isem[0]).wait()
                pltpu.make_async_copy(
                    dst_hbm.at[w, 0], dstb[nxt], isem[1]).wait()
        return ()

    lax.fori_loop(0, NBLK // 2, pair, ())
    plsc.subcore_barrier()
    pltpu.sync_copy(acc_sh.at[pl.ds(s * RPT, RPT)],
                    out_hbm.at[c, pl.ds(s * RPT, RPT)])


RB = 2560


def _dinv(d0, d1):
    return lax.rsqrt(d0 + d1 + 1.0)


def _prep_body(x_ref, w_ref, d0_ref, d1_ref, o_ref):
    dinv = _dinv(d0_ref[...], d1_ref[...])
    o_ref[...] = jnp.dot(x_ref[...], w_ref[...],
                         preferred_element_type=jnp.float32) * dinv


def _mid_body(a0_ref, a1_ref, hs_ref, d0_ref, d1_ref, b_ref, w_ref, o_ref):
    dinv = _dinv(d0_ref[...], d1_ref[...])
    t = (a0_ref[...] + a1_ref[...] + hs_ref[...]) * dinv + b_ref[...]
    x2 = jnp.maximum(t, 0.0)
    o_ref[...] = jnp.dot(x2, w_ref[...],
                         preferred_element_type=jnp.float32) * dinv


def _final_body(a0_ref, a1_ref, hs_ref, d0_ref, d1_ref, b_ref, o_ref):
    dinv = _dinv(d0_ref[...], d1_ref[...])
    t = (a0_ref[...] + a1_ref[...] + hs_ref[...]) * dinv + b_ref[...]
    o_ref[...] = jnp.maximum(t, 0.0)


_row_spec = pl.BlockSpec((RB, D), lambda i: (i, 0))
_col_spec = pl.BlockSpec((RB, 1), lambda i: (i, 0))
_mat_spec = pl.BlockSpec((D, D), lambda i: (0, 0))
_bias_spec = pl.BlockSpec((1, D), lambda i: (0, 0))
_out_row = jax.ShapeDtypeStruct((NP, D), jnp.float32)

_prep = pl.pallas_call(
    _prep_body,
    grid=(NP // RB,),
    in_specs=[_row_spec, _mat_spec, _col_spec, _col_spec],
    out_specs=_row_spec,
    out_shape=_out_row,
)

_mid = pl.pallas_call(
    _mid_body,
    grid=(NP // RB,),
    in_specs=[_row_spec, _row_spec, _row_spec, _col_spec, _col_spec,
              _bias_spec, _mat_spec],
    out_specs=_row_spec,
    out_shape=_out_row,
)

_final = pl.pallas_call(
    _final_body,
    grid=(NP // RB,),
    in_specs=[_row_spec, _row_spec, _row_spec, _col_spec, _col_spec,
              _bias_spec],
    out_specs=_row_spec,
    out_shape=_out_row,
)

_PAD_SRC = np.asarray(
    (np.arange(NW * PADW, dtype=np.int64) * 41) % N, np.int32
).reshape(NW, PADW)
_PAD_DST = np.asarray(
    N + (np.arange(NW * PADW, dtype=np.int64) * 7) % (NP - N), np.int32
).reshape(NW, PADW)


def kernel(x, edge_index, W1, b1, W2, b2):
    src_p = jnp.concatenate(
        [edge_index[0].reshape(NW, EPW), jnp.asarray(_PAD_SRC)], axis=1)
    dst_p = jnp.concatenate(
        [edge_index[1].reshape(NW, EPW), jnp.asarray(_PAD_DST)], axis=1)
    src_r = src_p.reshape(NW, NBLK, IBK, CH)
    dst_r = dst_p.reshape(NW, NBLK, IBK, CH)
    dst_deg = dst_p.reshape(NW, NCH, DCH)
    x_pad = jnp.concatenate(
        [x, jnp.zeros((NP - N, D), jnp.float32)], axis=0)

    degp = _deg_pass(dst_deg)
    d0 = degp[0].reshape(NP, 1)
    d1 = degp[1].reshape(NP, 1)

    hs1 = _prep(x_pad, W1, d0, d1)
    acc1 = _edge_pass(hs1, src_r, dst_r)
    hs2 = _mid(acc1[0], acc1[1], hs1, d0, d1, b1.reshape(1, D), W2)
    acc2 = _edge_pass(hs2, src_r, dst_r)
    out = _final(acc2[0], acc2[1], hs2, d0, d1, b2.reshape(1, D))
    return out[:N]
